# baseline (speedup 1.0000x reference)
import functools

import jax
import jax.numpy as jnp
from jax import lax
from jax.experimental import pallas as pl
from jax.experimental.pallas import tpu as pltpu
from jax.experimental.pallas import tpu_sc as plsc

N = 10000
E = 320000
D = 128
NEG_SLOPE = 0.2

NC = 2
NS = 16
L = 16
NT = NC * NS
EPT = E // NT

KA = 400
NCHA = EPT // KA

K = 80
NCH = EPT // K

ZR = 16
NRC = N // ZR

ROW_BLK = 2000
GRID = N // ROW_BLK


def _mm_body(x_ref, w_ref, aap_ref, msg_ref, alt_ref):
    m = jnp.dot(x_ref[...], w_ref[...], preferred_element_type=jnp.float32)
    msg_ref[...] = m
    alt_ref[...] = jnp.dot(m, aap_ref[...], preferred_element_type=jnp.float32)


_mm_call = pl.pallas_call(
    _mm_body,
    grid=(GRID,),
    in_specs=[
        pl.BlockSpec((ROW_BLK, D), lambda i: (i, 0)),
        pl.BlockSpec((D, D), lambda i: (0, 0)),
        pl.BlockSpec((D, 8), lambda i: (0, 0)),
    ],
    out_specs=[
        pl.BlockSpec((ROW_BLK, D), lambda i: (i, 0)),
        pl.BlockSpec((ROW_BLK, 8), lambda i: (i, 0)),
    ],
    out_shape=[
        jax.ShapeDtypeStruct((N, D), jnp.float32),
        jax.ShapeDtypeStruct((N, 8), jnp.float32),
    ],
)


def _sca_body(as_hbm, ad_hbm, i_hbm, j_hbm,
              w_out, den_out,
              as_v, ad_v, den_v, iv_a, jv_a, iv_b, jv_b, wv_a, wv_b,
              si_a, si_b, so_a, so_b):
    c = lax.axis_index("c")
    s = lax.axis_index("s")
    t = c * NS + s

    pltpu.sync_copy(as_hbm, as_v)
    pltpu.sync_copy(ad_hbm, ad_v)

    zero = jnp.zeros((L,), jnp.float32)

    @pl.loop(0, N // L)
    def _zero_den(r):
        den_v[pl.ds(r * L, L)] = zero

    def _issue_idx(g, iv, jv, sem):
        base = t * EPT + g * KA
        pltpu.async_copy(i_hbm.at[pl.ds(base, KA)], iv, sem)
        pltpu.async_copy(j_hbm.at[pl.ds(base, KA)], jv, sem)

    def _wait_idx(g, iv, jv, sem):
        base = t * EPT + g * KA
        pltpu.make_async_copy(i_hbm.at[pl.ds(base, KA)], iv, sem).wait()
        pltpu.make_async_copy(j_hbm.at[pl.ds(base, KA)], jv, sem).wait()

    def _compute(iv, jv, wv):
        for g in range(KA // L):
            sl = pl.ds(g * L, L)
            ivg = iv[sl]
            jvg = jv[sl]
            e = plsc.load_gather(as_v, [ivg]) + plsc.load_gather(ad_v, [jvg])
            e = jnp.where(e >= 0.0, e, e * NEG_SLOPE)
            w = jnp.exp(e)
            wv[sl] = w
            plsc.addupdate_scatter(den_v, [ivg], w)

    def _issue_w(g, wv, sem):
        pltpu.async_copy(wv, w_out.at[pl.ds(t * EPT + g * KA, KA)], sem)

    def _wait_w(g, wv, sem):
        pltpu.make_async_copy(wv, w_out.at[pl.ds(t * EPT + g * KA, KA)],
                              sem).wait()

    _issue_idx(0, iv_a, jv_a, si_a)
    _issue_idx(1, iv_b, jv_b, si_b)

    @pl.loop(0, NCHA - 1, step=2)
    def _pair(p):
        _wait_idx(p, iv_a, jv_a, si_a)
        _compute(iv_a, jv_a, wv_a)
        _issue_w(p, wv_a, so_a)
        _issue_idx(p + 2, iv_a, jv_a, si_a)
        _wait_idx(p + 1, iv_b, jv_b, si_b)
        _compute(iv_b, jv_b, wv_b)
        _issue_w(p + 1, wv_b, so_b)

        @pl.when(p + 3 < NCHA)
        def _():
            _issue_idx(p + 3, iv_b, jv_b, si_b)

        _wait_w(p, wv_a, so_a)
        _wait_w(p + 1, wv_b, so_b)

    _wait_idx(NCHA - 1, iv_a, jv_a, si_a)
    _compute(iv_a, jv_a, wv_a)
    _issue_w(NCHA - 1, wv_a, so_a)
    _wait_w(NCHA - 1, wv_a, so_a)

    pltpu.sync_copy(den_v, den_out.at[pl.ds(t * N, N)])


_sca_call = functools.partial(
    pl.kernel,
    out_type=[
        jax.ShapeDtypeStruct((E,), jnp.float32),
        jax.ShapeDtypeStruct((NT * N,), jnp.float32),
    ],
    mesh=plsc.VectorSubcoreMesh(core_axis_name="c", subcore_axis_name="s"),
    compiler_params=pltpu.CompilerParams(needs_layout_passes=False),
    scratch_types=[
        pltpu.VMEM((N,), jnp.float32),
        pltpu.VMEM((N,), jnp.float32),
        pltpu.VMEM((N,), jnp.float32),
        pltpu.VMEM((KA,), jnp.int32),
        pltpu.VMEM((KA,), jnp.int32),
        pltpu.VMEM((KA,), jnp.int32),
        pltpu.VMEM((KA,), jnp.int32),
        pltpu.VMEM((KA,), jnp.float32),
        pltpu.VMEM((KA,), jnp.float32),
        pltpu.SemaphoreType.DMA,
        pltpu.SemaphoreType.DMA,
        pltpu.SemaphoreType.DMA,
        pltpu.SemaphoreType.DMA,
    ],
)(_sca_body)


def _scb_body(msg_hbm, i_hbm, j_hbm, w_hbm,
              acc_out,
              iv0, jv0, wv0, iv1, jv1, wv1, iv2, jv2, wv2, iv3, jv3, wv3,
              rows_a, rows_b, rows_c,
              si0, si1, si2, si3, sg_a, sg_b, sg_c, ss_a, ss_b, ss_c,
              acc_s):
    c = lax.axis_index("c")
    s = lax.axis_index("s")
    t = c * NS + s

    ivs = [iv0, iv1, iv2, iv3]
    jvs = [jv0, jv1, jv2, jv3]
    wvs = [wv0, wv1, wv2, wv3]
    sis = [si0, si1, si2, si3]
    rowss = [rows_a, rows_b, rows_c]
    sgs = [sg_a, sg_b, sg_c]
    sss = [ss_a, ss_b, ss_c]

    zero = jnp.zeros((L,), jnp.float32)

    @pl.loop(0, ZR)
    def _zero_rows(r):
        for c8 in range(D // L):
            rows_a[r, pl.ds(c8 * L, L)] = zero

    @pl.loop(s, NRC, step=NS)
    def _zero_acc(k):
        pltpu.sync_copy(rows_a.at[pl.ds(0, ZR)], acc_s.at[pl.ds(k * ZR, ZR)])

    plsc.subcore_barrier()

    def _issue_in(g, q):
        base = t * EPT + g * K
        pltpu.async_copy(i_hbm.at[pl.ds(base, K)], ivs[q], sis[q])
        pltpu.async_copy(j_hbm.at[pl.ds(base, K)], jvs[q], sis[q])
        pltpu.async_copy(w_hbm.at[pl.ds(base, K)], wvs[q], sis[q])

    def _wait_in(g, q):
        base = t * EPT + g * K
        pltpu.make_async_copy(i_hbm.at[pl.ds(base, K)], ivs[q], sis[q]).wait()
        pltpu.make_async_copy(j_hbm.at[pl.ds(base, K)], jvs[q], sis[q]).wait()
        pltpu.make_async_copy(w_hbm.at[pl.ds(base, K)], wvs[q], sis[q]).wait()

    def _issue_gather(q, r):
        pltpu.async_copy(msg_hbm.at[jvs[q]], rowss[r], sgs[r])

    def _wait_gather(q, r):
        pltpu.make_async_copy(msg_hbm.at[jvs[q]], rowss[r], sgs[r]).wait()

    def _wait_scatter(q, r):
        pltpu.make_async_copy(rowss[r], acc_s.at[ivs[q]], sss[r]).wait()

    def _scale(q, r):
        wv = wvs[q]
        rows = rowss[r]

        @pl.loop(0, K, unroll=4)
        def _scale_rows(rr):
            wb = plsc.load_gather(wv, [jnp.full((L,), rr, jnp.int32)])
            for c8 in range(D // L):
                sl = pl.ds(c8 * L, L)
                rows[rr, sl] = rows[rr, sl] * wb

    def _issue_scatter(q, r):
        pltpu.async_copy(rowss[r], acc_s.at[ivs[q]], sss[r], add=True)

    def _step(g, q, r, first=False):
        q2 = (q + 2) % 4
        r2 = (r + 2) % 3
        _wait_in(g + 2, q2)
        if not first:
            _wait_scatter((q + 3) % 4, r2)
        _issue_gather(q2, r2)
        _wait_gather(q, r)
        _scale(q, r)
        _issue_scatter(q, r)

        @pl.when(g + 3 < NCH)
        def _():
            _issue_in(g + 3, (q + 3) % 4)

    _issue_in(0, 0)
    _issue_in(1, 1)
    _issue_in(2, 2)
    _wait_in(0, 0)
    _issue_gather(0, 0)
    _wait_in(1, 1)
    _issue_gather(1, 1)
    _step(0, 0, 0, first=True)
    _step(1, 1, 1)
    _step(2, 2, 2)

    @pl.loop(0, (NCH - 5) // 12)
    def _twelve(m):
        g0 = m * 12 + 3
        for j in range(12):
            _step(g0 + j, (3 + j) % 4, (3 + j) % 3)

    g = NCH - 2
    q, r = g % 4, g % 3
    _wait_scatter((q + 3) % 4, (r + 2) % 3)
    _wait_gather(q, r)
    _scale(q, r)
    _issue_scatter(q, r)
    g = NCH - 1
    q, r = g % 4, g % 3
    _wait_gather(q, r)
    _scale(q, r)
    _issue_scatter(q, r)
    _wait_scatter((g - 1) % 4, (g - 1) % 3)
    _wait_scatter(q, r)

    plsc.subcore_barrier()

    @pl.loop(s, NRC, step=NS)
    def _stripe_out(k):
        pltpu.sync_copy(acc_s.at[pl.ds(k * ZR, ZR)],
                        acc_out.at[c, pl.ds(k * ZR, ZR)])


_scb_call = functools.partial(
    pl.kernel,
    out_type=jax.ShapeDtypeStruct((NC, N, D), jnp.float32),
    mesh=plsc.VectorSubcoreMesh(core_axis_name="c", subcore_axis_name="s"),
    compiler_params=pltpu.CompilerParams(needs_layout_passes=False),
    scratch_types=(
        [pltpu.VMEM((K,), jnp.int32)] * 2 + [pltpu.VMEM((K,), jnp.float32)]
        + [pltpu.VMEM((K,), jnp.int32)] * 2 + [pltpu.VMEM((K,), jnp.float32)]
        + [pltpu.VMEM((K,), jnp.int32)] * 2 + [pltpu.VMEM((K,), jnp.float32)]
        + [pltpu.VMEM((K,), jnp.int32)] * 2 + [pltpu.VMEM((K,), jnp.float32)]
        + [
            pltpu.VMEM((K, D), jnp.float32),
            pltpu.VMEM((K, D), jnp.float32),
            pltpu.VMEM((K, D), jnp.float32),
            pltpu.SemaphoreType.DMA,
            pltpu.SemaphoreType.DMA,
            pltpu.SemaphoreType.DMA,
            pltpu.SemaphoreType.DMA,
            pltpu.SemaphoreType.DMA,
            pltpu.SemaphoreType.DMA,
            pltpu.SemaphoreType.DMA,
            pltpu.SemaphoreType.DMA,
            pltpu.SemaphoreType.DMA,
            pltpu.SemaphoreType.DMA,
            pltpu.VMEM_SHARED((N, D), jnp.float32),
        ]
    ),
)(_scb_body)


def _fin_body(acc_ref, den_ref, out_ref):
    a = acc_ref[0] + acc_ref[1]
    d = jnp.sum(den_ref[...], axis=1)
    d = jnp.where(d > 0.0, d, 1.0)
    out_ref[...] = jnp.maximum(a / d[:, None], 0.0)


_fin_call = pl.pallas_call(
    _fin_body,
    grid=(GRID,),
    in_specs=[
        pl.BlockSpec((NC, ROW_BLK, D), lambda i: (0, i, 0)),
        pl.BlockSpec((ROW_BLK, NT), lambda i: (i, 0)),
    ],
    out_specs=pl.BlockSpec((ROW_BLK, D), lambda i: (i, 0)),
    out_shape=jax.ShapeDtypeStruct((N, D), jnp.float32),
)


def kernel(x_source, edge_index, W, a):
    aap = jnp.zeros((D, 8), jnp.float32)
    aap = aap.at[:, 0].set(a[:D, 0]).at[:, 1].set(a[D:, 0])
    msg, alt = _mm_call(x_source, W, aap)
    i_arr = edge_index[0]
    j_arr = edge_index[1]
    w_e, den = _sca_call(alt[:, 0], alt[:, 1], i_arr, j_arr)
    acc = _scb_call(msg, i_arr, j_arr, w_e)
    return _fin_call(acc, den.reshape(NT, N).T)


# speedup vs baseline: 1.0122x; 1.0122x over previous
import functools

import jax
import jax.numpy as jnp
from jax import lax
from jax.experimental import pallas as pl
from jax.experimental.pallas import tpu as pltpu
from jax.experimental.pallas import tpu_sc as plsc

N = 10000
E = 320000
D = 128
NEG_SLOPE = 0.2

NC = 2
NS = 16
L = 16
NT = NC * NS
EPT = E // NT

KA = 400
NCHA = EPT // KA

K = 80
NCH = EPT // K

ZR = 16
NRC = N // ZR

ROW_BLK = 2000
GRID = N // ROW_BLK


def _mm_body(x_ref, w_ref, aap_ref, msg_ref, alt_ref):
    m = jnp.dot(x_ref[...], w_ref[...], preferred_element_type=jnp.float32)
    msg_ref[...] = m
    alt_ref[...] = jnp.dot(m, aap_ref[...], preferred_element_type=jnp.float32)


_mm_call = pl.pallas_call(
    _mm_body,
    grid=(GRID,),
    in_specs=[
        pl.BlockSpec((ROW_BLK, D), lambda i: (i, 0)),
        pl.BlockSpec((D, D), lambda i: (0, 0)),
        pl.BlockSpec((D, 8), lambda i: (0, 0)),
    ],
    out_specs=[
        pl.BlockSpec((ROW_BLK, D), lambda i: (i, 0)),
        pl.BlockSpec((ROW_BLK, 8), lambda i: (i, 0)),
    ],
    out_shape=[
        jax.ShapeDtypeStruct((N, D), jnp.float32),
        jax.ShapeDtypeStruct((N, 8), jnp.float32),
    ],
)


def _sca_body(as_hbm, ad_hbm, i_hbm, j_hbm,
              w_out, den_out,
              as_v, ad_v, den_v, iv_a, jv_a, iv_b, jv_b, wv_a, wv_b,
              si_a, si_b, so_a, so_b):
    c = lax.axis_index("c")
    s = lax.axis_index("s")
    t = c * NS + s

    pltpu.sync_copy(as_hbm, as_v)
    pltpu.sync_copy(ad_hbm, ad_v)

    zero = jnp.zeros((L,), jnp.float32)

    @pl.loop(0, N // L)
    def _zero_den(r):
        den_v[pl.ds(r * L, L)] = zero

    def _issue_idx(g, iv, jv, sem):
        base = t * EPT + g * KA
        pltpu.async_copy(i_hbm.at[pl.ds(base, KA)], iv, sem)
        pltpu.async_copy(j_hbm.at[pl.ds(base, KA)], jv, sem)

    def _wait_idx(g, iv, jv, sem):
        base = t * EPT + g * KA
        pltpu.make_async_copy(i_hbm.at[pl.ds(base, KA)], iv, sem).wait()
        pltpu.make_async_copy(j_hbm.at[pl.ds(base, KA)], jv, sem).wait()

    def _compute(iv, jv, wv):
        for g in range(KA // L):
            sl = pl.ds(g * L, L)
            ivg = iv[sl]
            jvg = jv[sl]
            e = plsc.load_gather(as_v, [ivg]) + plsc.load_gather(ad_v, [jvg])
            e = jnp.where(e >= 0.0, e, e * NEG_SLOPE)
            w = jnp.exp(e)
            wv[sl] = w
            plsc.addupdate_scatter(den_v, [ivg], w)

    def _issue_w(g, wv, sem):
        pltpu.async_copy(wv, w_out.at[pl.ds(t * EPT + g * KA, KA)], sem)

    def _wait_w(g, wv, sem):
        pltpu.make_async_copy(wv, w_out.at[pl.ds(t * EPT + g * KA, KA)],
                              sem).wait()

    _issue_idx(0, iv_a, jv_a, si_a)
    _issue_idx(1, iv_b, jv_b, si_b)

    @pl.loop(0, NCHA - 1, step=2)
    def _pair(p):
        _wait_idx(p, iv_a, jv_a, si_a)
        _compute(iv_a, jv_a, wv_a)
        _issue_w(p, wv_a, so_a)
        _issue_idx(p + 2, iv_a, jv_a, si_a)
        _wait_idx(p + 1, iv_b, jv_b, si_b)
        _compute(iv_b, jv_b, wv_b)
        _issue_w(p + 1, wv_b, so_b)

        @pl.when(p + 3 < NCHA)
        def _():
            _issue_idx(p + 3, iv_b, jv_b, si_b)

        _wait_w(p, wv_a, so_a)
        _wait_w(p + 1, wv_b, so_b)

    _wait_idx(NCHA - 1, iv_a, jv_a, si_a)
    _compute(iv_a, jv_a, wv_a)
    _issue_w(NCHA - 1, wv_a, so_a)
    _wait_w(NCHA - 1, wv_a, so_a)

    pltpu.sync_copy(den_v, den_out.at[pl.ds(t * N, N)])


_sca_call = functools.partial(
    pl.kernel,
    out_type=[
        jax.ShapeDtypeStruct((E,), jnp.float32),
        jax.ShapeDtypeStruct((NT * N,), jnp.float32),
    ],
    mesh=plsc.VectorSubcoreMesh(core_axis_name="c", subcore_axis_name="s"),
    compiler_params=pltpu.CompilerParams(needs_layout_passes=False),
    scratch_types=[
        pltpu.VMEM((N,), jnp.float32),
        pltpu.VMEM((N,), jnp.float32),
        pltpu.VMEM((N,), jnp.float32),
        pltpu.VMEM((KA,), jnp.int32),
        pltpu.VMEM((KA,), jnp.int32),
        pltpu.VMEM((KA,), jnp.int32),
        pltpu.VMEM((KA,), jnp.int32),
        pltpu.VMEM((KA,), jnp.float32),
        pltpu.VMEM((KA,), jnp.float32),
        pltpu.SemaphoreType.DMA,
        pltpu.SemaphoreType.DMA,
        pltpu.SemaphoreType.DMA,
        pltpu.SemaphoreType.DMA,
    ],
)(_sca_body)


def _scb_body(msg_hbm, i_hbm, j_hbm, w_hbm,
              acc_out,
              iv0, jv0, wv0, iv1, jv1, wv1, iv2, jv2, wv2, iv3, jv3, wv3,
              rows_a, rows_b,
              si0, si1, si2, si3, sg_a, sg_b, ss_a, ss_b,
              acc_s):
    c = lax.axis_index("c")
    s = lax.axis_index("s")
    t = c * NS + s

    ivs = [iv0, iv1, iv2, iv3]
    jvs = [jv0, jv1, jv2, jv3]
    wvs = [wv0, wv1, wv2, wv3]
    sis = [si0, si1, si2, si3]
    rowss = [rows_a, rows_b]
    sgs = [sg_a, sg_b]
    sss = [ss_a, ss_b]

    zero = jnp.zeros((L,), jnp.float32)

    @pl.loop(0, ZR)
    def _zero_rows(r):
        for c8 in range(D // L):
            rows_a[r, pl.ds(c8 * L, L)] = zero

    @pl.loop(s, NRC, step=NS)
    def _zero_acc(k):
        pltpu.sync_copy(rows_a.at[pl.ds(0, ZR)], acc_s.at[pl.ds(k * ZR, ZR)])

    plsc.subcore_barrier()

    def _issue_in(g, q):
        base = t * EPT + g * K
        pltpu.async_copy(i_hbm.at[pl.ds(base, K)], ivs[q], sis[q])
        pltpu.async_copy(j_hbm.at[pl.ds(base, K)], jvs[q], sis[q])
        pltpu.async_copy(w_hbm.at[pl.ds(base, K)], wvs[q], sis[q])

    def _wait_in(g, q):
        base = t * EPT + g * K
        pltpu.make_async_copy(i_hbm.at[pl.ds(base, K)], ivs[q], sis[q]).wait()
        pltpu.make_async_copy(j_hbm.at[pl.ds(base, K)], jvs[q], sis[q]).wait()
        pltpu.make_async_copy(w_hbm.at[pl.ds(base, K)], wvs[q], sis[q]).wait()

    def _issue_gather(q, r):
        pltpu.async_copy(msg_hbm.at[jvs[q]], rowss[r], sgs[r])

    def _wait_gather(q, r):
        pltpu.make_async_copy(msg_hbm.at[jvs[q]], rowss[r], sgs[r]).wait()

    def _wait_scatter(q, r):
        pltpu.make_async_copy(rowss[r], acc_s.at[ivs[q]], sss[r]).wait()

    def _scale(q, r):
        wv = wvs[q]
        rows = rowss[r]

        @pl.loop(0, K, unroll=4)
        def _scale_rows(rr):
            wb = plsc.load_gather(wv, [jnp.full((L,), rr, jnp.int32)])
            for c8 in range(D // L):
                sl = pl.ds(c8 * L, L)
                rows[rr, sl] = rows[rr, sl] * wb

    def _issue_scatter(q, r):
        pltpu.async_copy(rowss[r], acc_s.at[ivs[q]], sss[r], add=True)

    def _step(g, q, first=False, in2=True):
        q1 = (q + 1) % 4
        r = q % 2
        r1 = (q + 1) % 2
        _wait_in(g + 1, q1)
        if not first:
            _wait_scatter((q + 3) % 4, r1)
        _issue_gather(q1, r1)
        _wait_gather(q, r)
        _scale(q, r)
        _issue_scatter(q, r)
        if in2:
            _issue_in(g + 2, (q + 2) % 4)

    _issue_in(0, 0)
    _issue_in(1, 1)
    _wait_in(0, 0)
    _issue_gather(0, 0)
    _step(0, 0, first=True)

    @pl.loop(0, (NCH - 5) // 4)
    def _quad(m):
        g0 = m * 4 + 1
        for j in range(4):
            _step(g0 + j, (1 + j) % 4)

    _step(NCH - 4, (NCH - 4) % 4, in2=True)
    _step(NCH - 3, (NCH - 3) % 4, in2=True)
    _step(NCH - 2, (NCH - 2) % 4, in2=False)
    g = NCH - 1
    q = g % 4
    _wait_scatter((q + 3) % 4, (g - 1) % 2)
    _wait_gather(q, g % 2)
    _scale(q, g % 2)
    _issue_scatter(q, g % 2)
    _wait_scatter(q, g % 2)

    plsc.subcore_barrier()

    @pl.loop(s, NRC, step=NS)
    def _stripe_out(k):
        pltpu.sync_copy(acc_s.at[pl.ds(k * ZR, ZR)],
                        acc_out.at[c, pl.ds(k * ZR, ZR)])


_scb_call = functools.partial(
    pl.kernel,
    out_type=jax.ShapeDtypeStruct((NC, N, D), jnp.float32),
    mesh=plsc.VectorSubcoreMesh(core_axis_name="c", subcore_axis_name="s"),
    compiler_params=pltpu.CompilerParams(needs_layout_passes=False),
    scratch_types=(
        [pltpu.VMEM((K,), jnp.int32)] * 2 + [pltpu.VMEM((K,), jnp.float32)]
        + [pltpu.VMEM((K,), jnp.int32)] * 2 + [pltpu.VMEM((K,), jnp.float32)]
        + [pltpu.VMEM((K,), jnp.int32)] * 2 + [pltpu.VMEM((K,), jnp.float32)]
        + [pltpu.VMEM((K,), jnp.int32)] * 2 + [pltpu.VMEM((K,), jnp.float32)]
        + [
            pltpu.VMEM((K, D), jnp.float32),
            pltpu.VMEM((K, D), jnp.float32),
            pltpu.SemaphoreType.DMA,
            pltpu.SemaphoreType.DMA,
            pltpu.SemaphoreType.DMA,
            pltpu.SemaphoreType.DMA,
            pltpu.SemaphoreType.DMA,
            pltpu.SemaphoreType.DMA,
            pltpu.SemaphoreType.DMA,
            pltpu.SemaphoreType.DMA,
            pltpu.VMEM_SHARED((N, D), jnp.float32),
        ]
    ),
)(_scb_body)


def _fin_body(acc_ref, den_ref, out_ref):
    a = acc_ref[0] + acc_ref[1]
    d = jnp.sum(den_ref[...], axis=1)
    d = jnp.where(d > 0.0, d, 1.0)
    out_ref[...] = jnp.maximum(a / d[:, None], 0.0)


_fin_call = pl.pallas_call(
    _fin_body,
    grid=(GRID,),
    in_specs=[
        pl.BlockSpec((NC, ROW_BLK, D), lambda i: (0, i, 0)),
        pl.BlockSpec((ROW_BLK, NT), lambda i: (i, 0)),
    ],
    out_specs=pl.BlockSpec((ROW_BLK, D), lambda i: (i, 0)),
    out_shape=jax.ShapeDtypeStruct((N, D), jnp.float32),
)


def kernel(x_source, edge_index, W, a):
    aap = jnp.zeros((D, 8), jnp.float32)
    aap = aap.at[:, 0].set(a[:D, 0]).at[:, 1].set(a[D:, 0])
    msg, alt = _mm_call(x_source, W, aap)
    i_arr = edge_index[0]
    j_arr = edge_index[1]
    w_e, den = _sca_call(alt[:, 0], alt[:, 1], i_arr, j_arr)
    acc = _scb_call(msg, i_arr, j_arr, w_e)
    return _fin_call(acc, den.reshape(NT, N).T)
